# baseline (speedup 1.0000x reference)
import functools

import jax
import jax.numpy as jnp
from jax import lax
from jax.experimental import pallas as pl
from jax.experimental.pallas import tpu as pltpu
from jax.experimental.pallas import tpu_sc as plsc

_LANES = 16


def _sc_histograms(ei, et, n_nodes):
    B, _, E = ei.shape
    info = plsc.get_sparse_core_info()
    nc, ns = info.num_cores, info.num_subcores
    nw = nc * ns
    wpb = nw // B
    ch = E // wpb
    iters = ch // _LANES
    N = n_nodes

    def body(ei_hbm, et_hbm, z_hbm, out_hbm, src_v, dst_v, et_v, deg_h, h_h,
             res_v):
        wid = lax.axis_index("s") * nc + lax.axis_index("c")
        b = wid // wpb
        j = wid % wpb
        base = j * ch
        pltpu.sync_copy(ei_hbm.at[b, 0, pl.ds(base, ch)], src_v)
        pltpu.sync_copy(ei_hbm.at[b, 1, pl.ds(base, ch)], dst_v)
        pltpu.sync_copy(et_hbm.at[b, pl.ds(base, ch)], et_v)
        pltpu.sync_copy(z_hbm, deg_h)
        pltpu.sync_copy(z_hbm, h_h)
        lane_off = lax.iota(jnp.int32, _LANES) * N

        def edge_step(i, carry):
            off = i * _LANES
            s16 = src_v[pl.ds(off, _LANES)]
            d16 = dst_v[pl.ds(off, _LANES)]
            t16 = et_v[pl.ds(off, _LANES)]
            val = 1 + t16 * 65535
            plsc.addupdate_scatter(deg_h, [lane_off + d16], val)
            plsc.addupdate_scatter(h_h, [lane_off + s16], val, mask=d16 == 0)
            return carry

        lax.fori_loop(0, iters, edge_step, 0)

        def reduce_step(k, carry):
            off = k * _LANES
            a = deg_h[pl.ds(off, _LANES)]
            c = h_h[pl.ds(off, _LANES)]
            for rep in range(1, _LANES):
                a = a + deg_h[pl.ds(rep * N + off, _LANES)]
                c = c + h_h[pl.ds(rep * N + off, _LANES)]
            res_v[0, pl.ds(off, _LANES)] = a
            res_v[1, pl.ds(off, _LANES)] = c
            return carry

        lax.fori_loop(0, N // _LANES, reduce_step, 0)
        pltpu.sync_copy(res_v, out_hbm.at[b, j])

    zeros = jnp.zeros((_LANES * N,), jnp.int32)
    run = pl.kernel(
        body,
        out_type=jax.ShapeDtypeStruct((B, wpb, 2, N), jnp.int32),
        mesh=plsc.VectorSubcoreMesh(core_axis_name="c", subcore_axis_name="s"),
        compiler_params=pltpu.CompilerParams(needs_layout_passes=False),
        scratch_types=[
            pltpu.VMEM((ch,), jnp.int32),
            pltpu.VMEM((ch,), jnp.int32),
            pltpu.VMEM((ch,), jnp.int32),
            pltpu.VMEM((_LANES * N,), jnp.int32),
            pltpu.VMEM((_LANES * N,), jnp.int32),
            pltpu.VMEM((2, N), jnp.int32),
        ],
    )
    return run(ei, et, zeros)


def _ng_body(xf_ref, w_ref, out_ref):
    k = pl.program_id(0)

    @pl.when(k == 0)
    def _init():
        out_ref[...] = jnp.zeros_like(out_ref)

    out_ref[...] += jnp.dot(xf_ref[...], w_ref[...],
                            preferred_element_type=jnp.float32)


def _gate_matvec(xf, wg, k_chunk=32768):
    B, K = xf.shape
    D = wg.shape[1]
    nk = K // k_chunk
    return pl.pallas_call(
        _ng_body,
        grid=(nk,),
        in_specs=[
            pl.BlockSpec((B, k_chunk), lambda k: (0, k)),
            pl.BlockSpec((k_chunk, D), lambda k: (k, 0)),
        ],
        out_specs=pl.BlockSpec((B, D), lambda k: (0, 0)),
        out_shape=jax.ShapeDtypeStruct((B, D), jnp.float32),
    )(xf, wg)


def _state_body(R, cnt_ref, x_ref, ng_ref, wrel_ref, w0_ref, u_ref, x1_ref):
    B, wpb = cnt_ref.shape[0], cnt_ref.shape[1]
    D = x_ref.shape[2]
    m = jnp.zeros((1, D), jnp.float32)
    rows = []
    for b in range(B):
        acc = cnt_ref[b, 0]
        for j in range(1, wpb):
            acc = acc + cnt_ref[b, j]
        xb = x_ref[b]
        x0 = xb[0:1, :]
        p = jnp.dot(x0, w0_ref[...], preferred_element_type=jnp.float32)
        for r in range(R):
            dcnt = lax.shift_right_logical(acc[0:1, :], 16 * r) & 0xFFFF
            hcnt = lax.shift_right_logical(acc[1:2, :], 16 * r) & 0xFFFF
            deg = dcnt.astype(jnp.float32) + 1.0
            h = hcnt.astype(jnp.float32)
            d00 = deg[0:1, 0:1]
            cvec = h * lax.rsqrt(deg * d00)
            pre = jnp.dot(cvec, xb, preferred_element_type=jnp.float32)
            pre = pre + x0 / d00
            p = p + jnp.dot(pre, wrel_ref[r],
                            preferred_element_type=jnp.float32)
        gate = jax.nn.sigmoid(
            ng_ref[b:b + 1, :]
            + jnp.dot(m, u_ref[...], preferred_element_type=jnp.float32))
        m = gate * p + (1.0 - gate) * m
        rows.append(jnp.maximum(m, 0.0))
    x1_ref[...] = jnp.concatenate(rows, axis=0)


def _state(counts, x, ng, w_rel, w_0, u):
    B, _, D = x.shape
    R = w_rel.shape[0]
    return pl.pallas_call(
        functools.partial(_state_body, R),
        out_shape=jax.ShapeDtypeStruct((B, D), jnp.float32),
    )(counts, x, ng, w_rel, w_0, u)


def _proj_big_body(G, cg, x1_ref, w_ref, b_ref, out_ref, logits_v, rmax_v,
                   rsum_v):
    p = pl.program_id(0)
    c = pl.program_id(1)
    B = x1_ref.shape[0]

    @pl.when(p == 0)
    def _compute():
        l = lax.dot_general(x1_ref[...], w_ref[...],
                            (((1,), (1,)), ((), ())),
                            preferred_element_type=jnp.float32)
        l = l + b_ref[...]
        col = c * cg + lax.broadcasted_iota(jnp.int32, (B, cg), 1)
        l = jnp.where(col < G, l, -jnp.inf)
        logits_v[:, pl.ds(c * cg, cg)] = l
        mx = jnp.max(l, axis=1, keepdims=True)

        @pl.when(c == 0)
        def _first():
            rmax_v[...] = mx
            rsum_v[...] = jnp.sum(jnp.exp(l - mx), axis=1, keepdims=True)

        @pl.when(c > 0)
        def _rest():
            om = rmax_v[...]
            nm = jnp.maximum(om, mx)
            rsum_v[...] = (rsum_v[...] * jnp.exp(om - nm)
                           + jnp.sum(jnp.exp(l - nm), axis=1, keepdims=True))
            rmax_v[...] = nm

    @pl.when(p == 1)
    def _normalize():
        lse = rmax_v[...] + jnp.log(rsum_v[...])
        out_ref[...] = logits_v[:, pl.ds(c * cg, cg)] - lse


def _proj_logsoftmax_big(x1, w, bias2d, cg=2048):
    B, D = x1.shape
    G = w.shape[0]
    nch = (G + cg - 1) // cg
    return pl.pallas_call(
        functools.partial(_proj_big_body, G, cg),
        grid=(2, nch),
        in_specs=[
            pl.BlockSpec((B, D), lambda p, c: (0, 0)),
            pl.BlockSpec((cg, D), lambda p, c: (c * (1 - p), 0)),
            pl.BlockSpec((1, cg), lambda p, c: (0, c * (1 - p))),
        ],
        out_specs=pl.BlockSpec((B, cg), lambda p, c: (0, c)),
        out_shape=jax.ShapeDtypeStruct((B, G), jnp.float32),
        scratch_shapes=[
            pltpu.VMEM((B, nch * cg), jnp.float32),
            pltpu.VMEM((B, 1), jnp.float32),
            pltpu.VMEM((B, 1), jnp.float32),
        ],
    )(x1, w, bias2d)


def _proj_small_body(x1_ref, w_ref, b_ref, out_ref):
    l = lax.dot_general(x1_ref[...], w_ref[...], (((1,), (1,)), ((), ())),
                        preferred_element_type=jnp.float32)
    l = l + b_ref[...]
    mx = jnp.max(l, axis=1, keepdims=True)
    lse = mx + jnp.log(jnp.sum(jnp.exp(l - mx), axis=1, keepdims=True))
    out_ref[...] = l - lse


def _proj_logsoftmax_small(x1, w, bias2d):
    B = x1.shape[0]
    S = w.shape[0]
    return pl.pallas_call(
        _proj_small_body,
        out_shape=jax.ShapeDtypeStruct((B, S), jnp.float32),
    )(x1, w, bias2d)


def kernel(x, edge_index, edge_type, W_rel, W_0, update_gate_W,
           update_gate_U, lin_g_W, lin_g_b, lin_s_W, lin_s_b):
    B, N, D = x.shape
    G = lin_g_W.shape[0]
    S = lin_s_W.shape[0]
    ei = edge_index.astype(jnp.int32)
    et = edge_type.astype(jnp.int32)

    counts = _sc_histograms(ei, et, N)
    ng = _gate_matvec(x.reshape(B, N * D), update_gate_W)
    x1 = _state(counts, x, ng, W_rel, W_0, update_gate_U)
    preds_g = _proj_logsoftmax_big(x1, lin_g_W, lin_g_b.reshape(1, G))
    preds_s = _proj_logsoftmax_small(x1, lin_s_W, lin_s_b.reshape(1, S))
    return preds_g, preds_s


# speedup vs baseline: 1.1618x; 1.1618x over previous
import functools

import jax
import jax.numpy as jnp
from jax import lax
from jax.experimental import pallas as pl
from jax.experimental.pallas import tpu as pltpu
from jax.experimental.pallas import tpu_sc as plsc

_LANES = 16


def _sc_histograms(ei, et, n_nodes):
    B, _, E = ei.shape
    info = plsc.get_sparse_core_info()
    nc, ns = info.num_cores, info.num_subcores
    nw = nc * ns
    wpb = nw // B
    ch = E // wpb
    iters = ch // _LANES
    N = n_nodes

    def body(ei_hbm, et_hbm, z_hbm, out_hbm, src_v, dst_v, et_v, deg_h, h_h,
             res_v):
        wid = lax.axis_index("s") * nc + lax.axis_index("c")
        b = wid // wpb
        j = wid % wpb
        base = j * ch
        pltpu.sync_copy(ei_hbm.at[b, 0, pl.ds(base, ch)], src_v)
        pltpu.sync_copy(ei_hbm.at[b, 1, pl.ds(base, ch)], dst_v)
        pltpu.sync_copy(et_hbm.at[b, pl.ds(base, ch)], et_v)
        pltpu.sync_copy(z_hbm, deg_h)
        pltpu.sync_copy(z_hbm, h_h)
        lane_off = lax.iota(jnp.int32, _LANES) * N

        def edge_step(i, carry):
            off = i * _LANES
            s16 = src_v[pl.ds(off, _LANES)]
            d16 = dst_v[pl.ds(off, _LANES)]
            t16 = et_v[pl.ds(off, _LANES)]
            val = 1 + t16 * 65535
            plsc.addupdate_scatter(deg_h, [lane_off + d16], val)
            plsc.addupdate_scatter(h_h, [lane_off + s16], val, mask=d16 == 0)
            return carry

        lax.fori_loop(0, iters, edge_step, 0)

        def reduce_step(k, carry):
            off = k * _LANES
            a = deg_h[pl.ds(off, _LANES)]
            c = h_h[pl.ds(off, _LANES)]
            for rep in range(1, _LANES):
                a = a + deg_h[pl.ds(rep * N + off, _LANES)]
                c = c + h_h[pl.ds(rep * N + off, _LANES)]
            res_v[0, pl.ds(off, _LANES)] = a
            res_v[1, pl.ds(off, _LANES)] = c
            return carry

        lax.fori_loop(0, N // _LANES, reduce_step, 0)
        pltpu.sync_copy(res_v, out_hbm.at[b, j])

    zeros = jnp.zeros((_LANES * N,), jnp.int32)
    run = pl.kernel(
        body,
        out_type=jax.ShapeDtypeStruct((B, wpb, 2, N), jnp.int32),
        mesh=plsc.VectorSubcoreMesh(core_axis_name="c", subcore_axis_name="s"),
        compiler_params=pltpu.CompilerParams(needs_layout_passes=False),
        scratch_types=[
            pltpu.VMEM((ch,), jnp.int32),
            pltpu.VMEM((ch,), jnp.int32),
            pltpu.VMEM((ch,), jnp.int32),
            pltpu.VMEM((_LANES * N,), jnp.int32),
            pltpu.VMEM((_LANES * N,), jnp.int32),
            pltpu.VMEM((2, N), jnp.int32),
        ],
    )
    return run(ei, et, zeros)


def _ng_body(xf_ref, w_ref, out_ref):
    k = pl.program_id(0)

    @pl.when(k == 0)
    def _init():
        out_ref[...] = jnp.zeros_like(out_ref)

    out_ref[...] += jnp.dot(xf_ref[...], w_ref[...],
                            preferred_element_type=jnp.float32)


def _gate_matvec(xf, wg, k_chunk=32768):
    B, K = xf.shape
    D = wg.shape[1]
    nk = K // k_chunk
    return pl.pallas_call(
        _ng_body,
        grid=(nk,),
        in_specs=[
            pl.BlockSpec((B, k_chunk), lambda k: (0, k)),
            pl.BlockSpec((k_chunk, D), lambda k: (k, 0)),
        ],
        out_specs=pl.BlockSpec((B, D), lambda k: (0, 0)),
        out_shape=jax.ShapeDtypeStruct((B, D), jnp.float32),
    )(xf, wg)


def _state_body(R, cnt_ref, x_ref, ng_ref, wrel_ref, w0_ref, u_ref, x1_ref):
    B, wpb = cnt_ref.shape[0], cnt_ref.shape[1]
    D = x_ref.shape[2]
    m = jnp.zeros((1, D), jnp.float32)
    rows = []
    for b in range(B):
        acc = cnt_ref[b, 0]
        for j in range(1, wpb):
            acc = acc + cnt_ref[b, j]
        xb = x_ref[b]
        x0 = xb[0:1, :]
        p = jnp.dot(x0, w0_ref[...], preferred_element_type=jnp.float32)
        for r in range(R):
            dcnt = lax.shift_right_logical(acc[0:1, :], 16 * r) & 0xFFFF
            hcnt = lax.shift_right_logical(acc[1:2, :], 16 * r) & 0xFFFF
            deg = dcnt.astype(jnp.float32) + 1.0
            h = hcnt.astype(jnp.float32)
            d00 = deg[0:1, 0:1]
            cvec = h * lax.rsqrt(deg * d00)
            pre = jnp.dot(cvec, xb, preferred_element_type=jnp.float32)
            pre = pre + x0 / d00
            p = p + jnp.dot(pre, wrel_ref[r],
                            preferred_element_type=jnp.float32)
        gate = jax.nn.sigmoid(
            ng_ref[b:b + 1, :]
            + jnp.dot(m, u_ref[...], preferred_element_type=jnp.float32))
        m = gate * p + (1.0 - gate) * m
        rows.append(jnp.maximum(m, 0.0))
    x1_ref[...] = jnp.concatenate(rows, axis=0)


def _state(counts, x, ng, w_rel, w_0, u):
    B, _, D = x.shape
    R = w_rel.shape[0]
    return pl.pallas_call(
        functools.partial(_state_body, R),
        out_shape=jax.ShapeDtypeStruct((B, D), jnp.float32),
    )(counts, x, ng, w_rel, w_0, u)


def _proj_big_body(G, cg, x1_ref, w_ref, b_ref, out_ref, logits_v, rmax_v,
                   rsum_v):
    p = pl.program_id(0)
    c = pl.program_id(1)
    B = x1_ref.shape[0]

    @pl.when(p == 0)
    def _compute():
        l = lax.dot_general(x1_ref[...], w_ref[...],
                            (((1,), (1,)), ((), ())),
                            preferred_element_type=jnp.float32)
        l = l + b_ref[...]
        col = c * cg + lax.broadcasted_iota(jnp.int32, (B, cg), 1)
        l = jnp.where(col < G, l, -jnp.inf)
        logits_v[:, pl.ds(c * cg, cg)] = l
        mx = jnp.max(l, axis=1, keepdims=True)

        @pl.when(c == 0)
        def _first():
            rmax_v[...] = mx
            rsum_v[...] = jnp.sum(jnp.exp(l - mx), axis=1, keepdims=True)

        @pl.when(c > 0)
        def _rest():
            om = rmax_v[...]
            nm = jnp.maximum(om, mx)
            rsum_v[...] = (rsum_v[...] * jnp.exp(om - nm)
                           + jnp.sum(jnp.exp(l - nm), axis=1, keepdims=True))
            rmax_v[...] = nm

    @pl.when(p == 1)
    def _normalize():
        lse = rmax_v[...] + jnp.log(rsum_v[...])
        out_ref[...] = logits_v[:, pl.ds(c * cg, cg)] - lse


def _proj_logsoftmax_big(x1, w, bias2d, cg=8192):
    B, D = x1.shape
    G = w.shape[0]
    nch = (G + cg - 1) // cg
    return pl.pallas_call(
        functools.partial(_proj_big_body, G, cg),
        grid=(2, nch),
        in_specs=[
            pl.BlockSpec((B, D), lambda p, c: (0, 0)),
            pl.BlockSpec((cg, D), lambda p, c: (c * (1 - p), 0)),
            pl.BlockSpec((1, cg), lambda p, c: (0, c * (1 - p))),
        ],
        out_specs=pl.BlockSpec((B, cg), lambda p, c: (0, c)),
        out_shape=jax.ShapeDtypeStruct((B, G), jnp.float32),
        scratch_shapes=[
            pltpu.VMEM((B, nch * cg), jnp.float32),
            pltpu.VMEM((B, 1), jnp.float32),
            pltpu.VMEM((B, 1), jnp.float32),
        ],
    )(x1, w, bias2d)


def _proj_small_body(x1_ref, w_ref, b_ref, out_ref):
    l = lax.dot_general(x1_ref[...], w_ref[...], (((1,), (1,)), ((), ())),
                        preferred_element_type=jnp.float32)
    l = l + b_ref[...]
    mx = jnp.max(l, axis=1, keepdims=True)
    lse = mx + jnp.log(jnp.sum(jnp.exp(l - mx), axis=1, keepdims=True))
    out_ref[...] = l - lse


def _proj_logsoftmax_small(x1, w, bias2d):
    B = x1.shape[0]
    S = w.shape[0]
    return pl.pallas_call(
        _proj_small_body,
        out_shape=jax.ShapeDtypeStruct((B, S), jnp.float32),
    )(x1, w, bias2d)


def kernel(x, edge_index, edge_type, W_rel, W_0, update_gate_W,
           update_gate_U, lin_g_W, lin_g_b, lin_s_W, lin_s_b):
    B, N, D = x.shape
    G = lin_g_W.shape[0]
    S = lin_s_W.shape[0]
    ei = edge_index.astype(jnp.int32)
    et = edge_type.astype(jnp.int32)

    counts = _sc_histograms(ei, et, N)
    ng = _gate_matvec(x.reshape(B, N * D), update_gate_W)
    x1 = _state(counts, x, ng, W_rel, W_0, update_gate_U)
    preds_g = _proj_logsoftmax_big(x1, lin_g_W, lin_g_b.reshape(1, G))
    preds_s = _proj_logsoftmax_small(x1, lin_s_W, lin_s_b.reshape(1, S))
    return preds_g, preds_s


# speedup vs baseline: 1.1814x; 1.0169x over previous
import functools

import jax
import jax.numpy as jnp
from jax import lax
from jax.experimental import pallas as pl
from jax.experimental.pallas import tpu as pltpu
from jax.experimental.pallas import tpu_sc as plsc

_LANES = 16


def _sc_histograms(ei, et, n_nodes):
    B, _, E = ei.shape
    info = plsc.get_sparse_core_info()
    nc, ns = info.num_cores, info.num_subcores
    nw = nc * ns
    wpb = nw // B
    ch = E // wpb
    iters = ch // _LANES
    N = n_nodes

    def body(ei_hbm, et_hbm, z_hbm, out_hbm, src_v, dst_v, et_v, deg_h, h_h,
             res_v, sem):
        wid = lax.axis_index("s") * nc + lax.axis_index("c")
        b = wid // wpb
        j = wid % wpb
        base = j * ch
        cp1 = pltpu.async_copy(ei_hbm.at[b, 0, pl.ds(base, ch)], src_v, sem)
        cp2 = pltpu.async_copy(ei_hbm.at[b, 1, pl.ds(base, ch)], dst_v, sem)
        cp3 = pltpu.async_copy(et_hbm.at[b, pl.ds(base, ch)], et_v, sem)
        cp4 = pltpu.async_copy(z_hbm, deg_h, sem)
        cp5 = pltpu.async_copy(z_hbm, h_h, sem)
        cp1.wait()
        cp2.wait()
        cp3.wait()
        cp4.wait()
        cp5.wait()
        lane_off = lax.iota(jnp.int32, _LANES) * N

        def edge_step(i, carry):
            off = i * _LANES
            s16 = src_v[pl.ds(off, _LANES)]
            d16 = dst_v[pl.ds(off, _LANES)]
            t16 = et_v[pl.ds(off, _LANES)]
            val = 1 + t16 * 65535
            plsc.addupdate_scatter(deg_h, [lane_off + d16], val)
            plsc.addupdate_scatter(h_h, [lane_off + s16], val, mask=d16 == 0)
            return carry

        lax.fori_loop(0, iters, edge_step, 0)

        def reduce_step(k, carry):
            off = k * _LANES
            a = deg_h[pl.ds(off, _LANES)]
            c = h_h[pl.ds(off, _LANES)]
            for rep in range(1, _LANES):
                a = a + deg_h[pl.ds(rep * N + off, _LANES)]
                c = c + h_h[pl.ds(rep * N + off, _LANES)]
            res_v[0, pl.ds(off, _LANES)] = a
            res_v[1, pl.ds(off, _LANES)] = c
            return carry

        lax.fori_loop(0, N // _LANES, reduce_step, 0)
        pltpu.sync_copy(res_v, out_hbm.at[b, j])

    zeros = jnp.zeros((_LANES * N,), jnp.int32)
    run = pl.kernel(
        body,
        out_type=jax.ShapeDtypeStruct((B, wpb, 2, N), jnp.int32),
        mesh=plsc.VectorSubcoreMesh(core_axis_name="c", subcore_axis_name="s"),
        compiler_params=pltpu.CompilerParams(needs_layout_passes=False),
        scratch_types=[
            pltpu.VMEM((ch,), jnp.int32),
            pltpu.VMEM((ch,), jnp.int32),
            pltpu.VMEM((ch,), jnp.int32),
            pltpu.VMEM((_LANES * N,), jnp.int32),
            pltpu.VMEM((_LANES * N,), jnp.int32),
            pltpu.VMEM((2, N), jnp.int32),
            pltpu.SemaphoreType.DMA,
        ],
        cost_estimate=pl.CostEstimate(
            flops=4 * E * B, bytes_accessed=64 * 1024 * 1024,
            transcendentals=0),
    )
    return run(ei, et, zeros)


def _ng_body(xf_ref, w_ref, out_ref):
    k = pl.program_id(0)

    @pl.when(k == 0)
    def _init():
        out_ref[...] = jnp.zeros_like(out_ref)

    out_ref[...] += jnp.dot(xf_ref[...], w_ref[...],
                            preferred_element_type=jnp.float32)


def _gate_matvec(xf, wg, k_chunk=32768):
    B, K = xf.shape
    D = wg.shape[1]
    nk = K // k_chunk
    return pl.pallas_call(
        _ng_body,
        grid=(nk,),
        in_specs=[
            pl.BlockSpec((B, k_chunk), lambda k: (0, k)),
            pl.BlockSpec((k_chunk, D), lambda k: (k, 0)),
        ],
        out_specs=pl.BlockSpec((B, D), lambda k: (0, 0)),
        out_shape=jax.ShapeDtypeStruct((B, D), jnp.float32),
    )(xf, wg)


def _state_body(R, cnt_ref, x_ref, ng_ref, wrel_ref, w0_ref, u_ref, x1_ref):
    B, wpb = cnt_ref.shape[0], cnt_ref.shape[1]
    D = x_ref.shape[2]
    m = jnp.zeros((1, D), jnp.float32)
    rows = []
    for b in range(B):
        acc = cnt_ref[b, 0]
        for j in range(1, wpb):
            acc = acc + cnt_ref[b, j]
        xb = x_ref[b]
        x0 = xb[0:1, :]
        p = jnp.dot(x0, w0_ref[...], preferred_element_type=jnp.float32)
        for r in range(R):
            dcnt = lax.shift_right_logical(acc[0:1, :], 16 * r) & 0xFFFF
            hcnt = lax.shift_right_logical(acc[1:2, :], 16 * r) & 0xFFFF
            deg = dcnt.astype(jnp.float32) + 1.0
            h = hcnt.astype(jnp.float32)
            d00 = deg[0:1, 0:1]
            cvec = h * lax.rsqrt(deg * d00)
            pre = jnp.dot(cvec, xb, preferred_element_type=jnp.float32)
            pre = pre + x0 / d00
            p = p + jnp.dot(pre, wrel_ref[r],
                            preferred_element_type=jnp.float32)
        gate = jax.nn.sigmoid(
            ng_ref[b:b + 1, :]
            + jnp.dot(m, u_ref[...], preferred_element_type=jnp.float32))
        m = gate * p + (1.0 - gate) * m
        rows.append(jnp.maximum(m, 0.0))
    x1_ref[...] = jnp.concatenate(rows, axis=0)


def _state(counts, x, ng, w_rel, w_0, u):
    B, _, D = x.shape
    R = w_rel.shape[0]
    return pl.pallas_call(
        functools.partial(_state_body, R),
        out_shape=jax.ShapeDtypeStruct((B, D), jnp.float32),
    )(counts, x, ng, w_rel, w_0, u)


def _proj_big_body(G, cg, x1_ref, w_ref, b_ref, out_ref, logits_v, rmax_v,
                   rsum_v):
    p = pl.program_id(0)
    c = pl.program_id(1)
    B = x1_ref.shape[0]

    @pl.when(p == 0)
    def _compute():
        l = lax.dot_general(x1_ref[...], w_ref[...],
                            (((1,), (1,)), ((), ())),
                            preferred_element_type=jnp.float32)
        l = l + b_ref[...]
        col = c * cg + lax.broadcasted_iota(jnp.int32, (B, cg), 1)
        l = jnp.where(col < G, l, -jnp.inf)
        logits_v[:, pl.ds(c * cg, cg)] = l
        mx = jnp.max(l, axis=1, keepdims=True)

        @pl.when(c == 0)
        def _first():
            rmax_v[...] = mx
            rsum_v[...] = jnp.sum(jnp.exp(l - mx), axis=1, keepdims=True)

        @pl.when(c > 0)
        def _rest():
            om = rmax_v[...]
            nm = jnp.maximum(om, mx)
            rsum_v[...] = (rsum_v[...] * jnp.exp(om - nm)
                           + jnp.sum(jnp.exp(l - nm), axis=1, keepdims=True))
            rmax_v[...] = nm

    @pl.when(p == 1)
    def _normalize():
        lse = rmax_v[...] + jnp.log(rsum_v[...])
        out_ref[...] = logits_v[:, pl.ds(c * cg, cg)] - lse


def _proj_logsoftmax_big(x1, w, bias2d, cg=8192):
    B, D = x1.shape
    G = w.shape[0]
    nch = (G + cg - 1) // cg
    return pl.pallas_call(
        functools.partial(_proj_big_body, G, cg),
        grid=(2, nch),
        in_specs=[
            pl.BlockSpec((B, D), lambda p, c: (0, 0)),
            pl.BlockSpec((cg, D), lambda p, c: (c * (1 - p), 0)),
            pl.BlockSpec((1, cg), lambda p, c: (0, c * (1 - p))),
        ],
        out_specs=pl.BlockSpec((B, cg), lambda p, c: (0, c)),
        out_shape=jax.ShapeDtypeStruct((B, G), jnp.float32),
        scratch_shapes=[
            pltpu.VMEM((B, nch * cg), jnp.float32),
            pltpu.VMEM((B, 1), jnp.float32),
            pltpu.VMEM((B, 1), jnp.float32),
        ],
    )(x1, w, bias2d)


def _proj_small_body(x1_ref, w_ref, b_ref, out_ref):
    l = lax.dot_general(x1_ref[...], w_ref[...], (((1,), (1,)), ((), ())),
                        preferred_element_type=jnp.float32)
    l = l + b_ref[...]
    mx = jnp.max(l, axis=1, keepdims=True)
    lse = mx + jnp.log(jnp.sum(jnp.exp(l - mx), axis=1, keepdims=True))
    out_ref[...] = l - lse


def _proj_logsoftmax_small(x1, w, bias2d):
    B = x1.shape[0]
    S = w.shape[0]
    return pl.pallas_call(
        _proj_small_body,
        out_shape=jax.ShapeDtypeStruct((B, S), jnp.float32),
    )(x1, w, bias2d)


def kernel(x, edge_index, edge_type, W_rel, W_0, update_gate_W,
           update_gate_U, lin_g_W, lin_g_b, lin_s_W, lin_s_b):
    B, N, D = x.shape
    G = lin_g_W.shape[0]
    S = lin_s_W.shape[0]
    ei = edge_index.astype(jnp.int32)
    et = edge_type.astype(jnp.int32)

    counts = _sc_histograms(ei, et, N)
    ng = _gate_matvec(x.reshape(B, N * D), update_gate_W)
    x1 = _state(counts, x, ng, W_rel, W_0, update_gate_U)
    preds_g = _proj_logsoftmax_big(x1, lin_g_W, lin_g_b.reshape(1, G))
    preds_s = _proj_logsoftmax_small(x1, lin_s_W, lin_s_b.reshape(1, S))
    return preds_g, preds_s


# speedup vs baseline: 1.1996x; 1.0154x over previous
import functools

import jax
import jax.numpy as jnp
from jax import lax
from jax.experimental import pallas as pl
from jax.experimental.pallas import tpu as pltpu
from jax.experimental.pallas import tpu_sc as plsc

_LANES = 16


def _sc_histograms(ei, et, n_nodes):
    B, _, E = ei.shape
    info = plsc.get_sparse_core_info()
    nc, ns = info.num_cores, info.num_subcores
    nw = nc * ns
    wpb = nw // B
    ch = E // wpb
    iters = ch // _LANES
    N = n_nodes

    nrep = 8

    def body(ei_hbm, et_hbm, out_hbm, src_v, dst_v, et_v, deg_h, h_h,
             res_v, sem):
        wid = lax.axis_index("s") * nc + lax.axis_index("c")
        b = wid // wpb
        j = wid % wpb
        base = j * ch
        cp1 = pltpu.async_copy(ei_hbm.at[b, 0, pl.ds(base, ch)], src_v, sem)
        cp2 = pltpu.async_copy(ei_hbm.at[b, 1, pl.ds(base, ch)], dst_v, sem)
        cp3 = pltpu.async_copy(et_hbm.at[b, pl.ds(base, ch)], et_v, sem)

        zv = jnp.zeros((_LANES,), jnp.int32)

        def zero_step(k, carry):
            off = k * 4 * _LANES
            for u in range(4):
                deg_h[pl.ds(off + u * _LANES, _LANES)] = zv
                h_h[pl.ds(off + u * _LANES, _LANES)] = zv
            return carry

        lax.fori_loop(0, nrep * N // (4 * _LANES), zero_step, 0)
        cp1.wait()
        cp2.wait()
        cp3.wait()

        lanes = lax.iota(jnp.int32, _LANES)
        rep_off = (lanes & (nrep - 1)) * N
        lo = lanes < nrep
        hi = jnp.logical_not(lo)

        def edge_step(i, carry):
            off = i * _LANES
            s16 = src_v[pl.ds(off, _LANES)]
            d16 = dst_v[pl.ds(off, _LANES)]
            t16 = et_v[pl.ds(off, _LANES)]
            val = 1 + t16 * 65535
            di = rep_off + d16
            si = rep_off + s16
            z = d16 == 0
            plsc.addupdate_scatter(deg_h, [di], val, mask=lo)
            plsc.addupdate_scatter(deg_h, [di], val, mask=hi)
            plsc.addupdate_scatter(h_h, [si], val, mask=jnp.logical_and(z, lo))
            plsc.addupdate_scatter(h_h, [si], val, mask=jnp.logical_and(z, hi))
            return carry

        lax.fori_loop(0, iters, edge_step, 0)

        def reduce_step(k, carry):
            off = k * _LANES
            a = deg_h[pl.ds(off, _LANES)]
            c = h_h[pl.ds(off, _LANES)]
            for rep in range(1, nrep):
                a = a + deg_h[pl.ds(rep * N + off, _LANES)]
                c = c + h_h[pl.ds(rep * N + off, _LANES)]
            res_v[0, pl.ds(off, _LANES)] = a
            res_v[1, pl.ds(off, _LANES)] = c
            return carry

        lax.fori_loop(0, N // _LANES, reduce_step, 0)
        pltpu.sync_copy(res_v, out_hbm.at[b, j])

    run = pl.kernel(
        body,
        out_type=jax.ShapeDtypeStruct((B, wpb, 2, N), jnp.int32),
        mesh=plsc.VectorSubcoreMesh(core_axis_name="c", subcore_axis_name="s"),
        compiler_params=pltpu.CompilerParams(needs_layout_passes=False),
        scratch_types=[
            pltpu.VMEM((ch,), jnp.int32),
            pltpu.VMEM((ch,), jnp.int32),
            pltpu.VMEM((ch,), jnp.int32),
            pltpu.VMEM((8 * N,), jnp.int32),
            pltpu.VMEM((8 * N,), jnp.int32),
            pltpu.VMEM((2, N), jnp.int32),
            pltpu.SemaphoreType.DMA,
        ],
        cost_estimate=pl.CostEstimate(
            flops=4 * E * B, bytes_accessed=64 * 1024 * 1024,
            transcendentals=0),
    )
    return run(ei, et)


def _ng_body(xf_ref, w_ref, out_ref):
    k = pl.program_id(0)

    @pl.when(k == 0)
    def _init():
        out_ref[...] = jnp.zeros_like(out_ref)

    out_ref[...] += jnp.dot(xf_ref[...], w_ref[...],
                            preferred_element_type=jnp.float32)


def _gate_matvec(xf, wg, k_chunk=32768):
    B, K = xf.shape
    D = wg.shape[1]
    nk = K // k_chunk
    return pl.pallas_call(
        _ng_body,
        grid=(nk,),
        in_specs=[
            pl.BlockSpec((B, k_chunk), lambda k: (0, k)),
            pl.BlockSpec((k_chunk, D), lambda k: (k, 0)),
        ],
        out_specs=pl.BlockSpec((B, D), lambda k: (0, 0)),
        out_shape=jax.ShapeDtypeStruct((B, D), jnp.float32),
    )(xf, wg)


def _state_body(R, cnt_ref, x_ref, ng_ref, wrel_ref, w0_ref, u_ref, x1_ref):
    B, wpb = cnt_ref.shape[0], cnt_ref.shape[1]
    D = x_ref.shape[2]
    m = jnp.zeros((1, D), jnp.float32)
    rows = []
    for b in range(B):
        acc = cnt_ref[b, 0]
        for j in range(1, wpb):
            acc = acc + cnt_ref[b, j]
        xb = x_ref[b]
        x0 = xb[0:1, :]
        p = jnp.dot(x0, w0_ref[...], preferred_element_type=jnp.float32)
        for r in range(R):
            dcnt = lax.shift_right_logical(acc[0:1, :], 16 * r) & 0xFFFF
            hcnt = lax.shift_right_logical(acc[1:2, :], 16 * r) & 0xFFFF
            deg = dcnt.astype(jnp.float32) + 1.0
            h = hcnt.astype(jnp.float32)
            d00 = deg[0:1, 0:1]
            cvec = h * lax.rsqrt(deg * d00)
            pre = jnp.dot(cvec, xb, preferred_element_type=jnp.float32)
            pre = pre + x0 / d00
            p = p + jnp.dot(pre, wrel_ref[r],
                            preferred_element_type=jnp.float32)
        gate = jax.nn.sigmoid(
            ng_ref[b:b + 1, :]
            + jnp.dot(m, u_ref[...], preferred_element_type=jnp.float32))
        m = gate * p + (1.0 - gate) * m
        rows.append(jnp.maximum(m, 0.0))
    x1_ref[...] = jnp.concatenate(rows, axis=0)


def _state(counts, x, ng, w_rel, w_0, u):
    B, _, D = x.shape
    R = w_rel.shape[0]
    return pl.pallas_call(
        functools.partial(_state_body, R),
        out_shape=jax.ShapeDtypeStruct((B, D), jnp.float32),
    )(counts, x, ng, w_rel, w_0, u)


def _proj_big_body(G, cg, x1_ref, w_ref, b_ref, out_ref, logits_v, rmax_v,
                   rsum_v):
    p = pl.program_id(0)
    c = pl.program_id(1)
    B = x1_ref.shape[0]

    @pl.when(p == 0)
    def _compute():
        l = lax.dot_general(x1_ref[...], w_ref[...],
                            (((1,), (1,)), ((), ())),
                            preferred_element_type=jnp.float32)
        l = l + b_ref[...]
        col = c * cg + lax.broadcasted_iota(jnp.int32, (B, cg), 1)
        l = jnp.where(col < G, l, -jnp.inf)
        logits_v[:, pl.ds(c * cg, cg)] = l
        mx = jnp.max(l, axis=1, keepdims=True)

        @pl.when(c == 0)
        def _first():
            rmax_v[...] = mx
            rsum_v[...] = jnp.sum(jnp.exp(l - mx), axis=1, keepdims=True)

        @pl.when(c > 0)
        def _rest():
            om = rmax_v[...]
            nm = jnp.maximum(om, mx)
            rsum_v[...] = (rsum_v[...] * jnp.exp(om - nm)
                           + jnp.sum(jnp.exp(l - nm), axis=1, keepdims=True))
            rmax_v[...] = nm

    @pl.when(p == 1)
    def _normalize():
        lse = rmax_v[...] + jnp.log(rsum_v[...])
        out_ref[...] = logits_v[:, pl.ds(c * cg, cg)] - lse


def _proj_logsoftmax_big(x1, w, bias2d, cg=8192):
    B, D = x1.shape
    G = w.shape[0]
    nch = (G + cg - 1) // cg
    return pl.pallas_call(
        functools.partial(_proj_big_body, G, cg),
        grid=(2, nch),
        in_specs=[
            pl.BlockSpec((B, D), lambda p, c: (0, 0)),
            pl.BlockSpec((cg, D), lambda p, c: (c * (1 - p), 0)),
            pl.BlockSpec((1, cg), lambda p, c: (0, c * (1 - p))),
        ],
        out_specs=pl.BlockSpec((B, cg), lambda p, c: (0, c)),
        out_shape=jax.ShapeDtypeStruct((B, G), jnp.float32),
        scratch_shapes=[
            pltpu.VMEM((B, nch * cg), jnp.float32),
            pltpu.VMEM((B, 1), jnp.float32),
            pltpu.VMEM((B, 1), jnp.float32),
        ],
    )(x1, w, bias2d)


def _proj_small_body(x1_ref, w_ref, b_ref, out_ref):
    l = lax.dot_general(x1_ref[...], w_ref[...], (((1,), (1,)), ((), ())),
                        preferred_element_type=jnp.float32)
    l = l + b_ref[...]
    mx = jnp.max(l, axis=1, keepdims=True)
    lse = mx + jnp.log(jnp.sum(jnp.exp(l - mx), axis=1, keepdims=True))
    out_ref[...] = l - lse


def _proj_logsoftmax_small(x1, w, bias2d):
    B = x1.shape[0]
    S = w.shape[0]
    return pl.pallas_call(
        _proj_small_body,
        out_shape=jax.ShapeDtypeStruct((B, S), jnp.float32),
    )(x1, w, bias2d)


def kernel(x, edge_index, edge_type, W_rel, W_0, update_gate_W,
           update_gate_U, lin_g_W, lin_g_b, lin_s_W, lin_s_b):
    B, N, D = x.shape
    G = lin_g_W.shape[0]
    S = lin_s_W.shape[0]
    ei = edge_index.astype(jnp.int32)
    et = edge_type.astype(jnp.int32)

    counts = _sc_histograms(ei, et, N)
    ng = _gate_matvec(x.reshape(B, N * D), update_gate_W)
    x1 = _state(counts, x, ng, W_rel, W_0, update_gate_U)
    preds_g = _proj_logsoftmax_big(x1, lin_g_W, lin_g_b.reshape(1, G))
    preds_s = _proj_logsoftmax_small(x1, lin_s_W, lin_s_b.reshape(1, S))
    return preds_g, preds_s


# speedup vs baseline: 1.2383x; 1.0323x over previous
import functools

import jax
import jax.numpy as jnp
from jax import lax
from jax.experimental import pallas as pl
from jax.experimental.pallas import tpu as pltpu
from jax.experimental.pallas import tpu_sc as plsc

_LANES = 16


def _sc_histograms(ei, et, n_nodes):
    B, _, E = ei.shape
    info = plsc.get_sparse_core_info()
    nc, ns = info.num_cores, info.num_subcores
    nw = nc * ns
    wpb = nw // B
    ch = E // wpb
    iters = ch // _LANES
    N = n_nodes

    nrep = 8

    def body(ei_hbm, et_hbm, out_hbm, src_v, dst_v, et_v, deg_h, h_h,
             res_v, sem):
        wid = lax.axis_index("s") * nc + lax.axis_index("c")
        b = wid // wpb
        j = wid % wpb
        base = j * ch
        cp1 = pltpu.async_copy(ei_hbm.at[b, 0, pl.ds(base, ch)], src_v, sem)
        cp2 = pltpu.async_copy(ei_hbm.at[b, 1, pl.ds(base, ch)], dst_v, sem)
        cp3 = pltpu.async_copy(et_hbm.at[b, pl.ds(base, ch)], et_v, sem)

        zv = jnp.zeros((_LANES,), jnp.int32)

        def zero_step(k, carry):
            off = k * 4 * _LANES
            for u in range(4):
                deg_h[pl.ds(off + u * _LANES, _LANES)] = zv
                h_h[pl.ds(off + u * _LANES, _LANES)] = zv
            return carry

        lax.fori_loop(0, nrep * N // (4 * _LANES), zero_step, 0)
        cp1.wait()
        cp2.wait()
        cp3.wait()

        lanes = lax.iota(jnp.int32, _LANES)
        rep_off = (lanes & (nrep - 1)) * N
        lo = lanes < nrep
        hi = jnp.logical_not(lo)

        def edge_step(i, carry):
            off = i * _LANES
            s16 = src_v[pl.ds(off, _LANES)]
            d16 = dst_v[pl.ds(off, _LANES)]
            t16 = et_v[pl.ds(off, _LANES)]
            val = 1 + t16 * 65535
            di = rep_off + d16
            si = rep_off + s16
            z = d16 == 0
            plsc.addupdate_scatter(deg_h, [di], val, mask=lo)
            plsc.addupdate_scatter(deg_h, [di], val, mask=hi)
            plsc.addupdate_scatter(h_h, [si], val, mask=jnp.logical_and(z, lo))
            plsc.addupdate_scatter(h_h, [si], val, mask=jnp.logical_and(z, hi))
            return carry

        lax.fori_loop(0, iters, edge_step, 0)

        def reduce_step(k, carry):
            off = k * _LANES
            a = deg_h[pl.ds(off, _LANES)]
            c = h_h[pl.ds(off, _LANES)]
            for rep in range(1, nrep):
                a = a + deg_h[pl.ds(rep * N + off, _LANES)]
                c = c + h_h[pl.ds(rep * N + off, _LANES)]
            res_v[0, pl.ds(off, _LANES)] = a
            res_v[1, pl.ds(off, _LANES)] = c
            return carry

        lax.fori_loop(0, N // _LANES, reduce_step, 0)
        pltpu.sync_copy(res_v, out_hbm.at[b, j])

    run = pl.kernel(
        body,
        out_type=jax.ShapeDtypeStruct((B, wpb, 2, N), jnp.int32),
        mesh=plsc.VectorSubcoreMesh(core_axis_name="c", subcore_axis_name="s"),
        compiler_params=pltpu.CompilerParams(needs_layout_passes=False),
        scratch_types=[
            pltpu.VMEM((ch,), jnp.int32),
            pltpu.VMEM((ch,), jnp.int32),
            pltpu.VMEM((ch,), jnp.int32),
            pltpu.VMEM((8 * N,), jnp.int32),
            pltpu.VMEM((8 * N,), jnp.int32),
            pltpu.VMEM((2, N), jnp.int32),
            pltpu.SemaphoreType.DMA,
        ],
        cost_estimate=pl.CostEstimate(
            flops=4 * E * B, bytes_accessed=64 * 1024 * 1024,
            transcendentals=0),
    )
    return run(ei, et)


def _ng_body(xf_ref, w_ref, out_ref):
    k = pl.program_id(0)

    @pl.when(k == 0)
    def _init():
        out_ref[...] = jnp.zeros_like(out_ref)

    out_ref[...] += jnp.dot(xf_ref[...], w_ref[...],
                            preferred_element_type=jnp.float32)


def _gate_matvec(xf, wg, k_chunk=32768):
    B, K = xf.shape
    D = wg.shape[1]
    nk = K // k_chunk
    return pl.pallas_call(
        _ng_body,
        grid=(nk,),
        in_specs=[
            pl.BlockSpec((B, k_chunk), lambda k: (0, k)),
            pl.BlockSpec((k_chunk, D), lambda k: (k, 0)),
        ],
        out_specs=pl.BlockSpec((B, D), lambda k: (0, 0)),
        out_shape=jax.ShapeDtypeStruct((B, D), jnp.float32),
        compiler_params=pltpu.CompilerParams(
            dimension_semantics=("arbitrary",),
            vmem_limit_bytes=100 * 1024 * 1024),
    )(xf, wg)


def _state_compute(R, cnt_ref, x_ref, ng_ref, wrel_ref, w0_ref, u_ref):
    B, wpb = cnt_ref.shape[0], cnt_ref.shape[1]
    D = x_ref.shape[2]
    m = jnp.zeros((1, D), jnp.float32)
    rows = []
    for b in range(B):
        acc = cnt_ref[b, 0]
        for j in range(1, wpb):
            acc = acc + cnt_ref[b, j]
        xb = x_ref[b]
        x0 = xb[0:1, :]
        p = jnp.dot(x0, w0_ref[...], preferred_element_type=jnp.float32)
        for r in range(R):
            dcnt = lax.shift_right_logical(acc[0:1, :], 16 * r) & 0xFFFF
            hcnt = lax.shift_right_logical(acc[1:2, :], 16 * r) & 0xFFFF
            deg = dcnt.astype(jnp.float32) + 1.0
            h = hcnt.astype(jnp.float32)
            d00 = deg[0:1, 0:1]
            cvec = h * lax.rsqrt(deg * d00)
            pre = jnp.dot(cvec, xb, preferred_element_type=jnp.float32)
            pre = pre + x0 / d00
            p = p + jnp.dot(pre, wrel_ref[r],
                            preferred_element_type=jnp.float32)
        gate = jax.nn.sigmoid(
            ng_ref[b:b + 1, :]
            + jnp.dot(m, u_ref[...], preferred_element_type=jnp.float32))
        m = gate * p + (1.0 - gate) * m
        rows.append(jnp.maximum(m, 0.0))
    return jnp.concatenate(rows, axis=0)


def _tail_body(G, cg, R, cnt_ref, x_ref, ng_ref, wrel_ref, w0_ref, u_ref,
               ws_ref, bs_ref, wg_ref, bg_ref, outg_ref, outs_ref,
               x1_v, logits_v, rmax_v, rsum_v):
    p = pl.program_id(0)
    c = pl.program_id(1)
    B = x_ref.shape[0]

    @pl.when(jnp.logical_and(p == 0, c == 0))
    def _state_and_senses():
        x1 = _state_compute(R, cnt_ref, x_ref, ng_ref, wrel_ref, w0_ref,
                            u_ref)
        x1_v[...] = x1
        l = lax.dot_general(x1, ws_ref[...], (((1,), (1,)), ((), ())),
                            preferred_element_type=jnp.float32)
        l = l + bs_ref[...]
        mx = jnp.max(l, axis=1, keepdims=True)
        lse = mx + jnp.log(jnp.sum(jnp.exp(l - mx), axis=1, keepdims=True))
        outs_ref[...] = l - lse

    @pl.when(p == 0)
    def _chunk():
        l = lax.dot_general(x1_v[...], wg_ref[...],
                            (((1,), (1,)), ((), ())),
                            preferred_element_type=jnp.float32)
        l = l + bg_ref[...]
        col = c * cg + lax.broadcasted_iota(jnp.int32, (B, cg), 1)
        l = jnp.where(col < G, l, -jnp.inf)
        logits_v[:, pl.ds(c * cg, cg)] = l
        mx = jnp.max(l, axis=1, keepdims=True)

        @pl.when(c == 0)
        def _first():
            rmax_v[...] = mx
            rsum_v[...] = jnp.sum(jnp.exp(l - mx), axis=1, keepdims=True)

        @pl.when(c > 0)
        def _rest():
            om = rmax_v[...]
            nm = jnp.maximum(om, mx)
            rsum_v[...] = (rsum_v[...] * jnp.exp(om - nm)
                           + jnp.sum(jnp.exp(l - nm), axis=1, keepdims=True))
            rmax_v[...] = nm

    @pl.when(p == 1)
    def _normalize():
        lse = rmax_v[...] + jnp.log(rsum_v[...])
        outg_ref[...] = logits_v[:, pl.ds(c * cg, cg)] - lse


def _tail(counts, x, ng, w_rel, w_0, u, lin_g_W, bias_g2d, lin_s_W, bias_s2d,
          cg=8192):
    B, _, D = x.shape
    R = w_rel.shape[0]
    G = lin_g_W.shape[0]
    S = lin_s_W.shape[0]
    nch = (G + cg - 1) // cg
    return pl.pallas_call(
        functools.partial(_tail_body, G, cg, R),
        grid=(2, nch),
        in_specs=[
            pl.BlockSpec(counts.shape, lambda p, c: (0, 0, 0, 0)),
            pl.BlockSpec(x.shape, lambda p, c: (0, 0, 0)),
            pl.BlockSpec((B, D), lambda p, c: (0, 0)),
            pl.BlockSpec(w_rel.shape, lambda p, c: (0, 0, 0)),
            pl.BlockSpec((D, D), lambda p, c: (0, 0)),
            pl.BlockSpec((D, D), lambda p, c: (0, 0)),
            pl.BlockSpec((S, D), lambda p, c: (0, 0)),
            pl.BlockSpec((1, S), lambda p, c: (0, 0)),
            pl.BlockSpec((cg, D), lambda p, c: (c * (1 - p), 0)),
            pl.BlockSpec((1, cg), lambda p, c: (0, c * (1 - p))),
        ],
        out_specs=[
            pl.BlockSpec((B, cg), lambda p, c: (0, c)),
            pl.BlockSpec((B, S), lambda p, c: (0, 0)),
        ],
        out_shape=[
            jax.ShapeDtypeStruct((B, G), jnp.float32),
            jax.ShapeDtypeStruct((B, S), jnp.float32),
        ],
        scratch_shapes=[
            pltpu.VMEM((B, D), jnp.float32),
            pltpu.VMEM((B, nch * cg), jnp.float32),
            pltpu.VMEM((B, 1), jnp.float32),
            pltpu.VMEM((B, 1), jnp.float32),
        ],
    )(counts, x, ng, w_rel, w_0, u, lin_s_W, bias_s2d, lin_g_W, bias_g2d)


def kernel(x, edge_index, edge_type, W_rel, W_0, update_gate_W,
           update_gate_U, lin_g_W, lin_g_b, lin_s_W, lin_s_b):
    B, N, D = x.shape
    G = lin_g_W.shape[0]
    S = lin_s_W.shape[0]
    ei = edge_index.astype(jnp.int32)
    et = edge_type.astype(jnp.int32)

    counts = _sc_histograms(ei, et, N)
    ng = _gate_matvec(x.reshape(B, N * D), update_gate_W)
    preds_g, preds_s = _tail(counts, x, ng, W_rel, W_0, update_gate_U,
                             lin_g_W, lin_g_b.reshape(1, G),
                             lin_s_W, lin_s_b.reshape(1, S))
    return preds_g, preds_s


# speedup vs baseline: 1.2386x; 1.0003x over previous
import functools

import jax
import jax.numpy as jnp
from jax import lax
from jax.experimental import pallas as pl
from jax.experimental.pallas import tpu as pltpu
from jax.experimental.pallas import tpu_sc as plsc

_LANES = 16


def _sc_histograms(ei, et, n_nodes):
    B, _, E = ei.shape
    info = plsc.get_sparse_core_info()
    nc, ns = info.num_cores, info.num_subcores
    nw = nc * ns
    wpb = nw // B
    ch = E // wpb
    iters = ch // _LANES
    N = n_nodes

    nrep = 8

    def body(ei_hbm, et_hbm, out_hbm, src_v, dst_v, et_v, deg_h, h_h,
             res_v, sem):
        wid = lax.axis_index("s") * nc + lax.axis_index("c")
        b = wid // wpb
        j = wid % wpb
        base = j * ch
        cp1 = pltpu.async_copy(ei_hbm.at[b, 0, pl.ds(base, ch)], src_v, sem)
        cp2 = pltpu.async_copy(ei_hbm.at[b, 1, pl.ds(base, ch)], dst_v, sem)
        cp3 = pltpu.async_copy(et_hbm.at[b, pl.ds(base, ch)], et_v, sem)

        zv = jnp.zeros((_LANES,), jnp.int32)

        def zero_step(k, carry):
            off = k * 4 * _LANES
            for u in range(4):
                deg_h[pl.ds(off + u * _LANES, _LANES)] = zv
                h_h[pl.ds(off + u * _LANES, _LANES)] = zv
            return carry

        lax.fori_loop(0, nrep * N // (4 * _LANES), zero_step, 0)
        cp1.wait()
        cp2.wait()
        cp3.wait()

        lanes = lax.iota(jnp.int32, _LANES)
        rep_off = (lanes & (nrep - 1)) * N
        lo = lanes < nrep
        hi = jnp.logical_not(lo)

        def edge_step(i, carry):
            off = i * _LANES
            s16 = src_v[pl.ds(off, _LANES)]
            d16 = dst_v[pl.ds(off, _LANES)]
            t16 = et_v[pl.ds(off, _LANES)]
            val = 1 + t16 * 65535
            di = rep_off + d16
            si = rep_off + s16
            z = d16 == 0
            plsc.addupdate_scatter(deg_h, [di], val, mask=lo)
            plsc.addupdate_scatter(deg_h, [di], val, mask=hi)
            plsc.addupdate_scatter(h_h, [si], val, mask=jnp.logical_and(z, lo))
            plsc.addupdate_scatter(h_h, [si], val, mask=jnp.logical_and(z, hi))
            return carry

        lax.fori_loop(0, iters, edge_step, 0)

        def reduce_step(k, carry):
            off = k * _LANES
            a = deg_h[pl.ds(off, _LANES)]
            c = h_h[pl.ds(off, _LANES)]
            for rep in range(1, nrep):
                a = a + deg_h[pl.ds(rep * N + off, _LANES)]
                c = c + h_h[pl.ds(rep * N + off, _LANES)]
            res_v[0, pl.ds(off, _LANES)] = a
            res_v[1, pl.ds(off, _LANES)] = c
            return carry

        lax.fori_loop(0, N // _LANES, reduce_step, 0)
        pltpu.sync_copy(res_v, out_hbm.at[b, j])

    run = pl.kernel(
        body,
        out_type=jax.ShapeDtypeStruct((B, wpb, 2, N), jnp.int32),
        mesh=plsc.VectorSubcoreMesh(core_axis_name="c", subcore_axis_name="s"),
        compiler_params=pltpu.CompilerParams(needs_layout_passes=False,
                                             skip_device_barrier=True),
        scratch_types=[
            pltpu.VMEM((ch,), jnp.int32),
            pltpu.VMEM((ch,), jnp.int32),
            pltpu.VMEM((ch,), jnp.int32),
            pltpu.VMEM((8 * N,), jnp.int32),
            pltpu.VMEM((8 * N,), jnp.int32),
            pltpu.VMEM((2, N), jnp.int32),
            pltpu.SemaphoreType.DMA,
        ],
        cost_estimate=pl.CostEstimate(
            flops=4 * E * B, bytes_accessed=64 * 1024 * 1024,
            transcendentals=0),
    )
    return run(ei, et)


def _ng_body(xf_ref, w_ref, out_ref):
    k = pl.program_id(0)

    @pl.when(k == 0)
    def _init():
        out_ref[...] = jnp.zeros_like(out_ref)

    out_ref[...] += jnp.dot(xf_ref[...], w_ref[...],
                            preferred_element_type=jnp.float32)


def _gate_matvec(xf, wg, k_chunk=32768):
    B, K = xf.shape
    D = wg.shape[1]
    nk = K // k_chunk
    return pl.pallas_call(
        _ng_body,
        grid=(nk,),
        in_specs=[
            pl.BlockSpec((B, k_chunk), lambda k: (0, k)),
            pl.BlockSpec((k_chunk, D), lambda k: (k, 0)),
        ],
        out_specs=pl.BlockSpec((B, D), lambda k: (0, 0)),
        out_shape=jax.ShapeDtypeStruct((B, D), jnp.float32),
        compiler_params=pltpu.CompilerParams(
            dimension_semantics=("arbitrary",),
            vmem_limit_bytes=100 * 1024 * 1024),
    )(xf, wg)


def _state_compute(R, cnt_ref, x_ref, ng_ref, wrel_ref, w0_ref, u_ref):
    B, wpb = cnt_ref.shape[0], cnt_ref.shape[1]
    D = x_ref.shape[2]
    m = jnp.zeros((1, D), jnp.float32)
    rows = []
    for b in range(B):
        acc = cnt_ref[b, 0]
        for j in range(1, wpb):
            acc = acc + cnt_ref[b, j]
        xb = x_ref[b]
        x0 = xb[0:1, :]
        p = jnp.dot(x0, w0_ref[...], preferred_element_type=jnp.float32)
        for r in range(R):
            dcnt = lax.shift_right_logical(acc[0:1, :], 16 * r) & 0xFFFF
            hcnt = lax.shift_right_logical(acc[1:2, :], 16 * r) & 0xFFFF
            deg = dcnt.astype(jnp.float32) + 1.0
            h = hcnt.astype(jnp.float32)
            d00 = deg[0:1, 0:1]
            cvec = h * lax.rsqrt(deg * d00)
            pre = jnp.dot(cvec, xb, preferred_element_type=jnp.float32)
            pre = pre + x0 / d00
            p = p + jnp.dot(pre, wrel_ref[r],
                            preferred_element_type=jnp.float32)
        gate = jax.nn.sigmoid(
            ng_ref[b:b + 1, :]
            + jnp.dot(m, u_ref[...], preferred_element_type=jnp.float32))
        m = gate * p + (1.0 - gate) * m
        rows.append(jnp.maximum(m, 0.0))
    return jnp.concatenate(rows, axis=0)


def _tail_body(G, cg, R, cnt_ref, x_ref, ng_ref, wrel_ref, w0_ref, u_ref,
               ws_ref, bs_ref, wg_ref, bg_ref, outg_ref, outs_ref,
               x1_v, logits_v, rmax_v, rsum_v):
    p = pl.program_id(0)
    c = pl.program_id(1)
    B = x_ref.shape[0]

    @pl.when(jnp.logical_and(p == 0, c == 0))
    def _state_and_senses():
        x1 = _state_compute(R, cnt_ref, x_ref, ng_ref, wrel_ref, w0_ref,
                            u_ref)
        x1_v[...] = x1
        l = lax.dot_general(x1, ws_ref[...], (((1,), (1,)), ((), ())),
                            preferred_element_type=jnp.float32)
        l = l + bs_ref[...]
        mx = jnp.max(l, axis=1, keepdims=True)
        lse = mx + jnp.log(jnp.sum(jnp.exp(l - mx), axis=1, keepdims=True))
        outs_ref[...] = l - lse

    @pl.when(p == 0)
    def _chunk():
        l = lax.dot_general(x1_v[...], wg_ref[...],
                            (((1,), (1,)), ((), ())),
                            preferred_element_type=jnp.float32)
        l = l + bg_ref[...]
        col = c * cg + lax.broadcasted_iota(jnp.int32, (B, cg), 1)
        l = jnp.where(col < G, l, -jnp.inf)
        logits_v[:, pl.ds(c * cg, cg)] = l
        mx = jnp.max(l, axis=1, keepdims=True)

        @pl.when(c == 0)
        def _first():
            rmax_v[...] = mx
            rsum_v[...] = jnp.sum(jnp.exp(l - mx), axis=1, keepdims=True)

        @pl.when(c > 0)
        def _rest():
            om = rmax_v[...]
            nm = jnp.maximum(om, mx)
            rsum_v[...] = (rsum_v[...] * jnp.exp(om - nm)
                           + jnp.sum(jnp.exp(l - nm), axis=1, keepdims=True))
            rmax_v[...] = nm

    @pl.when(p == 1)
    def _normalize():
        lse = rmax_v[...] + jnp.log(rsum_v[...])
        outg_ref[...] = logits_v[:, pl.ds(c * cg, cg)] - lse


def _tail(counts, x, ng, w_rel, w_0, u, lin_g_W, bias_g2d, lin_s_W, bias_s2d,
          cg=8192):
    B, _, D = x.shape
    R = w_rel.shape[0]
    G = lin_g_W.shape[0]
    S = lin_s_W.shape[0]
    nch = (G + cg - 1) // cg
    return pl.pallas_call(
        functools.partial(_tail_body, G, cg, R),
        grid=(2, nch),
        in_specs=[
            pl.BlockSpec(counts.shape, lambda p, c: (0, 0, 0, 0)),
            pl.BlockSpec(x.shape, lambda p, c: (0, 0, 0)),
            pl.BlockSpec((B, D), lambda p, c: (0, 0)),
            pl.BlockSpec(w_rel.shape, lambda p, c: (0, 0, 0)),
            pl.BlockSpec((D, D), lambda p, c: (0, 0)),
            pl.BlockSpec((D, D), lambda p, c: (0, 0)),
            pl.BlockSpec((S, D), lambda p, c: (0, 0)),
            pl.BlockSpec((1, S), lambda p, c: (0, 0)),
            pl.BlockSpec((cg, D), lambda p, c: (c * (1 - p), 0)),
            pl.BlockSpec((1, cg), lambda p, c: (0, c * (1 - p))),
        ],
        out_specs=[
            pl.BlockSpec((B, cg), lambda p, c: (0, c)),
            pl.BlockSpec((B, S), lambda p, c: (0, 0)),
        ],
        out_shape=[
            jax.ShapeDtypeStruct((B, G), jnp.float32),
            jax.ShapeDtypeStruct((B, S), jnp.float32),
        ],
        scratch_shapes=[
            pltpu.VMEM((B, D), jnp.float32),
            pltpu.VMEM((B, nch * cg), jnp.float32),
            pltpu.VMEM((B, 1), jnp.float32),
            pltpu.VMEM((B, 1), jnp.float32),
        ],
    )(counts, x, ng, w_rel, w_0, u, lin_s_W, bias_s2d, lin_g_W, bias_g2d)


def kernel(x, edge_index, edge_type, W_rel, W_0, update_gate_W,
           update_gate_U, lin_g_W, lin_g_b, lin_s_W, lin_s_b):
    B, N, D = x.shape
    G = lin_g_W.shape[0]
    S = lin_s_W.shape[0]
    ei = edge_index.astype(jnp.int32)
    et = edge_type.astype(jnp.int32)

    counts = _sc_histograms(ei, et, N)
    ng = _gate_matvec(x.reshape(B, N * D), update_gate_W)
    preds_g, preds_s = _tail(counts, x, ng, W_rel, W_0, update_gate_U,
                             lin_g_W, lin_g_b.reshape(1, G),
                             lin_s_W, lin_s_b.reshape(1, S))
    return preds_g, preds_s


# speedup vs baseline: 1.2586x; 1.0161x over previous
import functools

import jax
import jax.numpy as jnp
from jax import lax
from jax.experimental import pallas as pl
from jax.experimental.pallas import tpu as pltpu
from jax.experimental.pallas import tpu_sc as plsc

_LANES = 16


def _sc_histograms(ei, et, n_nodes):
    B, _, E = ei.shape
    info = plsc.get_sparse_core_info()
    nc, ns = 1, info.num_subcores
    nw = nc * ns
    wpb = nw // B
    ch = E // wpb
    iters = ch // _LANES
    N = n_nodes

    nrep = 8

    def body(ei_hbm, et_hbm, out_hbm, src_v, dst_v, et_v, deg_h, h_h,
             res_v, sem):
        wid = lax.axis_index("s") * nc + lax.axis_index("c")
        b = wid // wpb
        j = wid % wpb
        base = j * ch
        cp1 = pltpu.async_copy(ei_hbm.at[b, 0, pl.ds(base, ch)], src_v, sem)
        cp2 = pltpu.async_copy(ei_hbm.at[b, 1, pl.ds(base, ch)], dst_v, sem)
        cp3 = pltpu.async_copy(et_hbm.at[b, pl.ds(base, ch)], et_v, sem)

        zv = jnp.zeros((_LANES,), jnp.int32)

        def zero_step(k, carry):
            off = k * 4 * _LANES
            for u in range(4):
                deg_h[pl.ds(off + u * _LANES, _LANES)] = zv
                h_h[pl.ds(off + u * _LANES, _LANES)] = zv
            return carry

        lax.fori_loop(0, nrep * N // (4 * _LANES), zero_step, 0)
        cp1.wait()
        cp2.wait()
        cp3.wait()

        lanes = lax.iota(jnp.int32, _LANES)
        rep_off = (lanes & (nrep - 1)) * N
        lo = lanes < nrep
        hi = jnp.logical_not(lo)

        def edge_step(i, carry):
            off = i * _LANES
            s16 = src_v[pl.ds(off, _LANES)]
            d16 = dst_v[pl.ds(off, _LANES)]
            t16 = et_v[pl.ds(off, _LANES)]
            val = 1 + t16 * 65535
            di = rep_off + d16
            si = rep_off + s16
            z = d16 == 0
            plsc.addupdate_scatter(deg_h, [di], val, mask=lo)
            plsc.addupdate_scatter(deg_h, [di], val, mask=hi)
            plsc.addupdate_scatter(h_h, [si], val, mask=jnp.logical_and(z, lo))
            plsc.addupdate_scatter(h_h, [si], val, mask=jnp.logical_and(z, hi))
            return carry

        lax.fori_loop(0, iters, edge_step, 0)

        def reduce_step(k, carry):
            off = k * _LANES
            a = deg_h[pl.ds(off, _LANES)]
            c = h_h[pl.ds(off, _LANES)]
            for rep in range(1, nrep):
                a = a + deg_h[pl.ds(rep * N + off, _LANES)]
                c = c + h_h[pl.ds(rep * N + off, _LANES)]
            res_v[0, pl.ds(off, _LANES)] = a
            res_v[1, pl.ds(off, _LANES)] = c
            return carry

        lax.fori_loop(0, N // _LANES, reduce_step, 0)
        pltpu.sync_copy(res_v, out_hbm.at[b, j])

    run = pl.kernel(
        body,
        out_type=jax.ShapeDtypeStruct((B, wpb, 2, N), jnp.int32),
        mesh=plsc.VectorSubcoreMesh(core_axis_name="c", subcore_axis_name="s",
                                    num_cores=nc),
        compiler_params=pltpu.CompilerParams(needs_layout_passes=False),
        scratch_types=[
            pltpu.VMEM((ch,), jnp.int32),
            pltpu.VMEM((ch,), jnp.int32),
            pltpu.VMEM((ch,), jnp.int32),
            pltpu.VMEM((8 * N,), jnp.int32),
            pltpu.VMEM((8 * N,), jnp.int32),
            pltpu.VMEM((2, N), jnp.int32),
            pltpu.SemaphoreType.DMA,
        ],
        cost_estimate=pl.CostEstimate(
            flops=4 * E * B, bytes_accessed=64 * 1024 * 1024,
            transcendentals=0),
    )
    return run(ei, et)


def _ng_body(xf_ref, w_ref, out_ref):
    k = pl.program_id(0)

    @pl.when(k == 0)
    def _init():
        out_ref[...] = jnp.zeros_like(out_ref)

    out_ref[...] += jnp.dot(xf_ref[...], w_ref[...],
                            preferred_element_type=jnp.float32)


def _gate_matvec(xf, wg, k_chunk=32768):
    B, K = xf.shape
    D = wg.shape[1]
    nk = K // k_chunk
    return pl.pallas_call(
        _ng_body,
        grid=(nk,),
        in_specs=[
            pl.BlockSpec((B, k_chunk), lambda k: (0, k)),
            pl.BlockSpec((k_chunk, D), lambda k: (k, 0)),
        ],
        out_specs=pl.BlockSpec((B, D), lambda k: (0, 0)),
        out_shape=jax.ShapeDtypeStruct((B, D), jnp.float32),
        compiler_params=pltpu.CompilerParams(
            dimension_semantics=("arbitrary",),
            vmem_limit_bytes=100 * 1024 * 1024),
    )(xf, wg)


def _state_compute(R, cnt_ref, x_ref, ng_ref, wrel_ref, w0_ref, u_ref):
    B, wpb = cnt_ref.shape[0], cnt_ref.shape[1]
    D = x_ref.shape[2]
    m = jnp.zeros((1, D), jnp.float32)
    rows = []
    for b in range(B):
        acc = cnt_ref[b, 0]
        for j in range(1, wpb):
            acc = acc + cnt_ref[b, j]
        xb = x_ref[b]
        x0 = xb[0:1, :]
        p = jnp.dot(x0, w0_ref[...], preferred_element_type=jnp.float32)
        for r in range(R):
            dcnt = lax.shift_right_logical(acc[0:1, :], 16 * r) & 0xFFFF
            hcnt = lax.shift_right_logical(acc[1:2, :], 16 * r) & 0xFFFF
            deg = dcnt.astype(jnp.float32) + 1.0
            h = hcnt.astype(jnp.float32)
            d00 = deg[0:1, 0:1]
            cvec = h * lax.rsqrt(deg * d00)
            pre = jnp.dot(cvec, xb, preferred_element_type=jnp.float32)
            pre = pre + x0 / d00
            p = p + jnp.dot(pre, wrel_ref[r],
                            preferred_element_type=jnp.float32)
        gate = jax.nn.sigmoid(
            ng_ref[b:b + 1, :]
            + jnp.dot(m, u_ref[...], preferred_element_type=jnp.float32))
        m = gate * p + (1.0 - gate) * m
        rows.append(jnp.maximum(m, 0.0))
    return jnp.concatenate(rows, axis=0)


def _tail_body(G, cg, R, cnt_ref, x_ref, ng_ref, wrel_ref, w0_ref, u_ref,
               ws_ref, bs_ref, wg_ref, bg_ref, outg_ref, outs_ref,
               x1_v, logits_v, rmax_v, rsum_v):
    p = pl.program_id(0)
    c = pl.program_id(1)
    B = x_ref.shape[0]

    @pl.when(jnp.logical_and(p == 0, c == 0))
    def _state_and_senses():
        x1 = _state_compute(R, cnt_ref, x_ref, ng_ref, wrel_ref, w0_ref,
                            u_ref)
        x1_v[...] = x1
        l = lax.dot_general(x1, ws_ref[...], (((1,), (1,)), ((), ())),
                            preferred_element_type=jnp.float32)
        l = l + bs_ref[...]
        mx = jnp.max(l, axis=1, keepdims=True)
        lse = mx + jnp.log(jnp.sum(jnp.exp(l - mx), axis=1, keepdims=True))
        outs_ref[...] = l - lse

    @pl.when(p == 0)
    def _chunk():
        l = lax.dot_general(x1_v[...], wg_ref[...],
                            (((1,), (1,)), ((), ())),
                            preferred_element_type=jnp.float32)
        l = l + bg_ref[...]
        col = c * cg + lax.broadcasted_iota(jnp.int32, (B, cg), 1)
        l = jnp.where(col < G, l, -jnp.inf)
        logits_v[:, pl.ds(c * cg, cg)] = l
        mx = jnp.max(l, axis=1, keepdims=True)

        @pl.when(c == 0)
        def _first():
            rmax_v[...] = mx
            rsum_v[...] = jnp.sum(jnp.exp(l - mx), axis=1, keepdims=True)

        @pl.when(c > 0)
        def _rest():
            om = rmax_v[...]
            nm = jnp.maximum(om, mx)
            rsum_v[...] = (rsum_v[...] * jnp.exp(om - nm)
                           + jnp.sum(jnp.exp(l - nm), axis=1, keepdims=True))
            rmax_v[...] = nm

    @pl.when(p == 1)
    def _normalize():
        lse = rmax_v[...] + jnp.log(rsum_v[...])
        outg_ref[...] = logits_v[:, pl.ds(c * cg, cg)] - lse


def _tail(counts, x, ng, w_rel, w_0, u, lin_g_W, bias_g2d, lin_s_W, bias_s2d,
          cg=8192):
    B, _, D = x.shape
    R = w_rel.shape[0]
    G = lin_g_W.shape[0]
    S = lin_s_W.shape[0]
    nch = (G + cg - 1) // cg
    return pl.pallas_call(
        functools.partial(_tail_body, G, cg, R),
        grid=(2, nch),
        in_specs=[
            pl.BlockSpec(counts.shape, lambda p, c: (0, 0, 0, 0)),
            pl.BlockSpec(x.shape, lambda p, c: (0, 0, 0)),
            pl.BlockSpec((B, D), lambda p, c: (0, 0)),
            pl.BlockSpec(w_rel.shape, lambda p, c: (0, 0, 0)),
            pl.BlockSpec((D, D), lambda p, c: (0, 0)),
            pl.BlockSpec((D, D), lambda p, c: (0, 0)),
            pl.BlockSpec((S, D), lambda p, c: (0, 0)),
            pl.BlockSpec((1, S), lambda p, c: (0, 0)),
            pl.BlockSpec((cg, D), lambda p, c: (c * (1 - p), 0)),
            pl.BlockSpec((1, cg), lambda p, c: (0, c * (1 - p))),
        ],
        out_specs=[
            pl.BlockSpec((B, cg), lambda p, c: (0, c)),
            pl.BlockSpec((B, S), lambda p, c: (0, 0)),
        ],
        out_shape=[
            jax.ShapeDtypeStruct((B, G), jnp.float32),
            jax.ShapeDtypeStruct((B, S), jnp.float32),
        ],
        scratch_shapes=[
            pltpu.VMEM((B, D), jnp.float32),
            pltpu.VMEM((B, nch * cg), jnp.float32),
            pltpu.VMEM((B, 1), jnp.float32),
            pltpu.VMEM((B, 1), jnp.float32),
        ],
    )(counts, x, ng, w_rel, w_0, u, lin_s_W, bias_s2d, lin_g_W, bias_g2d)


def kernel(x, edge_index, edge_type, W_rel, W_0, update_gate_W,
           update_gate_U, lin_g_W, lin_g_b, lin_s_W, lin_s_b):
    B, N, D = x.shape
    G = lin_g_W.shape[0]
    S = lin_s_W.shape[0]
    ei = edge_index.astype(jnp.int32)
    et = edge_type.astype(jnp.int32)

    counts = _sc_histograms(ei, et, N)
    ng = _gate_matvec(x.reshape(B, N * D), update_gate_W)
    preds_g, preds_s = _tail(counts, x, ng, W_rel, W_0, update_gate_U,
                             lin_g_W, lin_g_b.reshape(1, G),
                             lin_s_W, lin_s_b.reshape(1, S))
    return preds_g, preds_s
